# baseline (speedup 1.0000x reference)
import functools

import jax
import jax.numpy as jnp
from jax import lax
from jax.experimental import pallas as pl
from jax.experimental.pallas import tpu as pltpu
from jax.experimental.pallas import tpu_sc as plsc

_NUM_LEVELS = 8
_RESOLUTION = 32
_TABLE_SIZE = 32768
_FEATURE_DIM = 8
_N_POS = 786432
_OUT_DIM = _NUM_LEVELS * _FEATURE_DIM

_NUM_WORKERS = 32
_PER_WORKER = _N_POS // _NUM_WORKERS
_CHUNK = 768
_NUM_CHUNKS = _PER_WORKER // _CHUNK
_GROWS = 128
_NUM_GATHERS = _CHUNK // _GROWS
_LANES = 16

_mesh = plsc.VectorSubcoreMesh(core_axis_name="c", subcore_axis_name="s")


@functools.partial(
    pl.kernel,
    mesh=_mesh,
    compiler_params=pltpu.CompilerParams(use_tc_tiling_on_sc=False),
    out_type=jax.ShapeDtypeStruct((_N_POS, _OUT_DIM), jnp.float32),
    scratch_types=[
        pltpu.VMEM((2, 3, _CHUNK), jnp.float32),
        pltpu.VMEM((2, _NUM_GATHERS, _GROWS), jnp.int32),
        pltpu.VMEM((2, _CHUNK, _OUT_DIM), jnp.float32),
        pltpu.SemaphoreType.DMA,
        pltpu.SemaphoreType.DMA,
        pltpu.SemaphoreType.DMA,
    ],
)
def _encode(pos_hbm, table_hbm, out_hbm, pos_v, idx_v, rows_v, psem, gsem, osem):
    wid = lax.axis_index("s") * 2 + lax.axis_index("c")
    w_base = wid * _PER_WORKER

    def base(c):
        return w_base + c * _CHUNK

    def pos_copy(slot, b):
        return pltpu.make_async_copy(
            pos_hbm.at[:, pl.ds(b, _CHUNK)], pos_v.at[slot], psem)

    def out_copy(slot, b):
        return pltpu.make_async_copy(
            rows_v.at[slot], out_hbm.at[pl.ds(b, _CHUNK)], osem)

    def gather_copy(slot, g):
        return pltpu.make_async_copy(
            table_hbm.at[idx_v.at[slot, g]],
            rows_v.at[slot, pl.ds(g * _GROWS, _GROWS)],
            gsem)

    def gather_fire(slot):
        for g in range(_NUM_GATHERS):
            gather_copy(slot, g).start()

    def gather_wait(slot):
        for g in range(_NUM_GATHERS):
            gather_copy(slot, g).wait()

    def compute_idx(slot):
        for g in range(_NUM_GATHERS):
            for v in range(_GROWS // _LANES):
                s = g * _GROWS + v * _LANES
                x = pos_v[slot, 0, pl.ds(s, _LANES)]
                y = pos_v[slot, 1, pl.ds(s, _LANES)]
                z = pos_v[slot, 2, pl.ds(s, _LANES)]
                fx = jnp.clip((x + 1.0) * 0.5 * _RESOLUTION, 0.0, _RESOLUTION - 1)
                fy = jnp.clip((y + 1.0) * 0.5 * _RESOLUTION, 0.0, _RESOLUTION - 1)
                fz = jnp.clip((z + 1.0) * 0.5 * _RESOLUTION, 0.0, _RESOLUTION - 1)
                f = fx * float(_RESOLUTION * _RESOLUTION) + fy * float(_RESOLUTION) + fz
                idx_v[slot, g, pl.ds(v * _LANES, _LANES)] = f.astype(jnp.int32)

    def steady_step(c, p):
        pos_copy(1 - p, 0).wait()
        compute_idx(1 - p)
        pos_copy(p, base(c + 2)).start()
        out_copy(1 - p, 0).wait()
        gather_wait(p)
        out_copy(p, base(c)).start()
        gather_fire(1 - p)

    pos_copy(0, base(0)).start()
    pos_copy(0, 0).wait()
    compute_idx(0)
    pos_copy(1, base(1)).start()
    gather_fire(0)
    pos_copy(1, 0).wait()
    compute_idx(1)
    pos_copy(0, base(2)).start()
    gather_wait(0)
    out_copy(0, base(0)).start()
    gather_fire(1)

    def loop_body(i, carry):
        c = 1 + 2 * i
        steady_step(c, 1)
        steady_step(c + 1, 0)
        return carry

    lax.fori_loop(0, (_NUM_CHUNKS - 4) // 2, loop_body, 0)

    c = _NUM_CHUNKS - 3
    pos_copy(0, 0).wait()
    compute_idx(0)
    pos_copy(1, base(_NUM_CHUNKS - 1)).start()
    out_copy(0, 0).wait()
    gather_wait(1)
    out_copy(1, base(c)).start()
    gather_fire(0)

    c = _NUM_CHUNKS - 2
    pos_copy(1, 0).wait()
    compute_idx(1)
    out_copy(1, 0).wait()
    gather_wait(0)
    out_copy(0, base(c)).start()
    gather_fire(1)

    c = _NUM_CHUNKS - 1
    out_copy(0, 0).wait()
    gather_wait(1)
    out_copy(1, base(c)).start()
    out_copy(1, 0).wait()


def kernel(positions, tables):
    fused = jnp.transpose(tables, (1, 0, 2)).reshape(_TABLE_SIZE, _OUT_DIM)
    pos_t = jnp.transpose(positions)
    return _encode(pos_t, fused)


# speedup vs baseline: 1.0250x; 1.0250x over previous
import functools

import jax
import jax.numpy as jnp
from jax import lax
from jax.experimental import pallas as pl
from jax.experimental.pallas import tpu as pltpu
from jax.experimental.pallas import tpu_sc as plsc

_NUM_LEVELS = 8
_RESOLUTION = 32
_TABLE_SIZE = 32768
_FEATURE_DIM = 8
_N_POS = 786432
_OUT_DIM = _NUM_LEVELS * _FEATURE_DIM

_NUM_WORKERS = 32
_PER_WORKER = _N_POS // _NUM_WORKERS
_CHUNK = 768
_NUM_CHUNKS = _PER_WORKER // _CHUNK
_GROWS = 128
_NUM_GATHERS = _CHUNK // _GROWS
_LANES = 16

_mesh = plsc.VectorSubcoreMesh(core_axis_name="c", subcore_axis_name="s")


@functools.partial(
    pl.kernel,
    mesh=_mesh,
    compiler_params=pltpu.CompilerParams(use_tc_tiling_on_sc=False),
    out_type=jax.ShapeDtypeStruct((_N_POS, _OUT_DIM), jnp.float32),
    scratch_types=[
        pltpu.VMEM((2, 3, _CHUNK), jnp.float32),
        pltpu.VMEM((2, _NUM_GATHERS, _GROWS), jnp.int32),
        pltpu.VMEM((2, _CHUNK, _OUT_DIM), jnp.float32),
        pltpu.SemaphoreType.DMA,
        pltpu.SemaphoreType.DMA,
        pltpu.SemaphoreType.DMA,
    ],
)
def _encode(xs_hbm, ys_hbm, zs_hbm, table_hbm, out_hbm, pos_v, idx_v, rows_v, psem, gsem, osem):
    wid = lax.axis_index("s") * 2 + lax.axis_index("c")
    w_base = wid * _PER_WORKER

    def base(c):
        return w_base + c * _CHUNK

    def pos_copies(slot, b):
        return [
            pltpu.make_async_copy(
                src.at[pl.ds(b, _CHUNK)], pos_v.at[slot, d], psem)
            for d, src in enumerate((xs_hbm, ys_hbm, zs_hbm))
        ]

    def pos_start(slot, b):
        for cp in pos_copies(slot, b):
            cp.start()

    def pos_wait(slot):
        for cp in pos_copies(slot, 0):
            cp.wait()

    def out_copy(slot, b):
        return pltpu.make_async_copy(
            rows_v.at[slot], out_hbm.at[pl.ds(b, _CHUNK)], osem)

    def gather_copy(slot, g):
        return pltpu.make_async_copy(
            table_hbm.at[idx_v.at[slot, g]],
            rows_v.at[slot, pl.ds(g * _GROWS, _GROWS)],
            gsem)

    def gather_fire(slot):
        for g in range(_NUM_GATHERS):
            gather_copy(slot, g).start()

    def gather_wait(slot):
        for g in range(_NUM_GATHERS):
            gather_copy(slot, g).wait()

    def compute_idx(slot):
        for g in range(_NUM_GATHERS):
            for v in range(_GROWS // _LANES):
                s = g * _GROWS + v * _LANES
                x = pos_v[slot, 0, pl.ds(s, _LANES)]
                y = pos_v[slot, 1, pl.ds(s, _LANES)]
                z = pos_v[slot, 2, pl.ds(s, _LANES)]
                fx = jnp.clip((x + 1.0) * 0.5 * _RESOLUTION, 0.0, _RESOLUTION - 1)
                fy = jnp.clip((y + 1.0) * 0.5 * _RESOLUTION, 0.0, _RESOLUTION - 1)
                fz = jnp.clip((z + 1.0) * 0.5 * _RESOLUTION, 0.0, _RESOLUTION - 1)
                f = fx * float(_RESOLUTION * _RESOLUTION) + fy * float(_RESOLUTION) + fz
                idx_v[slot, g, pl.ds(v * _LANES, _LANES)] = f.astype(jnp.int32)

    def steady_step(c, p):
        pos_wait(1 - p)
        compute_idx(1 - p)
        pos_start(p, base(c + 2))
        out_copy(1 - p, 0).wait()
        gather_wait(p)
        out_copy(p, base(c)).start()
        gather_fire(1 - p)

    pos_start(0, base(0))
    pos_wait(0)
    compute_idx(0)
    pos_start(1, base(1))
    gather_fire(0)
    pos_wait(1)
    compute_idx(1)
    pos_start(0, base(2))
    gather_wait(0)
    out_copy(0, base(0)).start()
    gather_fire(1)

    def loop_body(i, carry):
        c = 1 + 2 * i
        steady_step(c, 1)
        steady_step(c + 1, 0)
        return carry

    lax.fori_loop(0, (_NUM_CHUNKS - 4) // 2, loop_body, 0)

    c = _NUM_CHUNKS - 3
    pos_wait(0)
    compute_idx(0)
    pos_start(1, base(_NUM_CHUNKS - 1))
    out_copy(0, 0).wait()
    gather_wait(1)
    out_copy(1, base(c)).start()
    gather_fire(0)

    c = _NUM_CHUNKS - 2
    pos_wait(1)
    compute_idx(1)
    out_copy(1, 0).wait()
    gather_wait(0)
    out_copy(0, base(c)).start()
    gather_fire(1)

    c = _NUM_CHUNKS - 1
    out_copy(0, 0).wait()
    gather_wait(1)
    out_copy(1, base(c)).start()
    out_copy(1, 0).wait()


def kernel(positions, tables):
    fused = jnp.transpose(tables, (1, 0, 2)).reshape(_TABLE_SIZE, _OUT_DIM)
    xs = positions[:, 0]
    ys = positions[:, 1]
    zs = positions[:, 2]
    return _encode(xs, ys, zs, fused)


# speedup vs baseline: 1.0356x; 1.0104x over previous
import functools

import jax
import jax.numpy as jnp
from jax import lax
from jax.experimental import pallas as pl
from jax.experimental.pallas import tpu as pltpu
from jax.experimental.pallas import tpu_sc as plsc

_NUM_LEVELS = 8
_RESOLUTION = 32
_TABLE_SIZE = 32768
_FEATURE_DIM = 8
_N_POS = 786432
_OUT_DIM = _NUM_LEVELS * _FEATURE_DIM

_NUM_WORKERS = 32
_PER_WORKER = _N_POS // _NUM_WORKERS
_CHUNK = 768
_NUM_CHUNKS = _PER_WORKER // _CHUNK
_GROWS = 128
_NUM_GATHERS = _CHUNK // _GROWS
_LANES = 16

_mesh = plsc.VectorSubcoreMesh(core_axis_name="c", subcore_axis_name="s")


@functools.partial(
    pl.kernel,
    mesh=_mesh,
    compiler_params=pltpu.CompilerParams(use_tc_tiling_on_sc=False),
    out_type=jax.ShapeDtypeStruct((_N_POS, _OUT_DIM), jnp.float32),
    scratch_types=[
        pltpu.VMEM((2, 3, _CHUNK), jnp.float32),
        pltpu.VMEM((2, _NUM_GATHERS, _GROWS), jnp.int32),
        pltpu.VMEM((2, _CHUNK, _OUT_DIM), jnp.float32),
        pltpu.SemaphoreType.DMA,
        pltpu.SemaphoreType.DMA,
        pltpu.SemaphoreType.DMA,
    ],
)
def _encode(xs_hbm, ys_hbm, zs_hbm, table_hbm, out_hbm,
            pos_v, idx_v, rows_v, psem, gsem, osem):
    wid = lax.axis_index("s") * 2 + lax.axis_index("c")
    w_base = wid * _PER_WORKER

    def base(c):
        return w_base + c * _CHUNK

    def pos_copies(slot, b):
        return [
            pltpu.make_async_copy(
                src.at[pl.ds(b, _CHUNK)], pos_v.at[slot, d], psem)
            for d, src in enumerate((xs_hbm, ys_hbm, zs_hbm))
        ]

    def pos_start(slot, b):
        for cp in pos_copies(slot, b):
            cp.start()

    def pos_wait(slot):
        for cp in pos_copies(slot, 0):
            cp.wait()

    def out_copy(slot, b):
        return pltpu.make_async_copy(
            rows_v.at[slot], out_hbm.at[pl.ds(b, _CHUNK)], osem)

    def gather_copy(slot, g):
        return pltpu.make_async_copy(
            table_hbm.at[idx_v.at[slot, g]],
            rows_v.at[slot, pl.ds(g * _GROWS, _GROWS)],
            gsem)

    def gather_fire(slot):
        for g in range(_NUM_GATHERS):
            gather_copy(slot, g).start()

    def gather_wait(slot):
        for g in range(_NUM_GATHERS):
            gather_copy(slot, g).wait()

    def compute_idx(slot):
        for g in range(_NUM_GATHERS):
            for v in range(_GROWS // _LANES):
                s = g * _GROWS + v * _LANES
                x = pos_v[slot, 0, pl.ds(s, _LANES)]
                y = pos_v[slot, 1, pl.ds(s, _LANES)]
                z = pos_v[slot, 2, pl.ds(s, _LANES)]
                fx = (x + 1.0) * 0.5 * _RESOLUTION
                fy = (y + 1.0) * 0.5 * _RESOLUTION
                fz = (z + 1.0) * 0.5 * _RESOLUTION
                f = fx * float(_RESOLUTION * _RESOLUTION) + fy * float(_RESOLUTION) + fz
                idx_v[slot, g, pl.ds(v * _LANES, _LANES)] = f.astype(jnp.int32)

    def steady_step(c, p):
        pos_wait(1 - p)
        compute_idx(1 - p)
        pos_start(p, base(c + 2))
        out_copy(1 - p, 0).wait()
        gather_fire(1 - p)
        gather_wait(p)
        out_copy(p, base(c)).start()

    pos_start(0, base(0))
    pos_wait(0)
    compute_idx(0)
    pos_start(1, base(1))
    gather_fire(0)
    pos_wait(1)
    compute_idx(1)
    pos_start(0, base(2))
    gather_fire(1)
    gather_wait(0)
    out_copy(0, base(0)).start()

    def loop_body(i, carry):
        c = 1 + 2 * i
        steady_step(c, 1)
        steady_step(c + 1, 0)
        return carry

    lax.fori_loop(0, (_NUM_CHUNKS - 4) // 2, loop_body, 0)

    c = _NUM_CHUNKS - 3
    pos_wait(0)
    compute_idx(0)
    pos_start(1, base(_NUM_CHUNKS - 1))
    out_copy(0, 0).wait()
    gather_fire(0)
    gather_wait(1)
    out_copy(1, base(c)).start()

    c = _NUM_CHUNKS - 2
    pos_wait(1)
    compute_idx(1)
    out_copy(1, 0).wait()
    gather_fire(1)
    gather_wait(0)
    out_copy(0, base(c)).start()

    c = _NUM_CHUNKS - 1
    out_copy(0, 0).wait()
    gather_wait(1)
    out_copy(1, base(c)).start()
    out_copy(1, 0).wait()


def kernel(positions, tables):
    fused = jnp.transpose(tables, (1, 0, 2)).reshape(_TABLE_SIZE, _OUT_DIM)
    pc = jnp.clip(positions, -1.0, 0.9375)
    xs = pc[:, 0]
    ys = pc[:, 1]
    zs = pc[:, 2]
    return _encode(xs, ys, zs, fused)
